# baseline (speedup 1.0000x reference)
import functools

import jax
import jax.numpy as jnp
from jax import lax
from jax.experimental import pallas as pl
from jax.experimental.pallas import tpu as pltpu
from jax.experimental.pallas import tpu_sc as plsc

POS_OFFSET = 2
LANES = 16


@functools.lru_cache(maxsize=None)
def _make_kernel(B, S, V, H, NC, NS):
    NW = NC * NS
    PB = 2
    PAIRS = B // PB
    WPP = NW // PAIRS
    SW = S // WPP
    C = 4
    NCH = SW // C
    H2 = H // 2
    HR = 2 * C
    ROWS = PB * HR
    NIDX = NCH * ROWS
    S2 = 2 * S
    vecs = H2 // LANES
    NBUF = 3

    mesh = plsc.VectorSubcoreMesh(core_axis_name="c", subcore_axis_name="s")

    @functools.partial(
        pl.kernel,
        mesh=mesh,
        out_type=jax.ShapeDtypeStruct((B * S2, H2), jnp.float32),
        scratch_types=[
            pltpu.VMEM((NIDX,), jnp.int32),
            *[pltpu.VMEM((LANES,), jnp.int32) for _ in range(NBUF)],
            *[pltpu.VMEM((ROWS, H2), jnp.float32) for _ in range(NBUF)],
            *[pltpu.VMEM((HR, H2), jnp.float32) for _ in range(NBUF)],
            *[pltpu.SemaphoreType.DMA for _ in range(3 * NBUF)],
        ],
    )
    def emb(ids_hbm, tok_hbm, pos_hbm, out_hbm, idx_v,
            pidx0, pidx1, pidx2, tok0, tok1, tok2, pos0, pos1, pos2,
            ts0, ts1, ts2, ps0, ps1, ps2, ss0, ss1, ss2):
        wid = lax.axis_index("s") * NC + lax.axis_index("c")
        pair = wid // WPP
        s0 = (wid % WPP) * SW
        pltpu.sync_copy(ids_hbm.at[pl.ds(wid * NIDX, NIDX)], idx_v)

        pidx = (pidx0, pidx1, pidx2)
        toks = (tok0, tok1, tok2)
        poss = (pos0, pos1, pos2)
        tsem = (ts0, ts1, ts2)
        psem = (ps0, ps1, ps2)
        ssem = (ss0, ss1, ss2)

        def tok_desc(g, m):
            return pltpu.make_async_copy(
                tok_hbm.at[idx_v.at[pl.ds(g * ROWS, ROWS)]], toks[m], tsem[m]
            )

        def pos_desc(g, m):
            return pltpu.make_async_copy(
                pos_hbm.at[pidx[m].at[pl.ds(0, HR)]], poss[m], psem[m]
            )

        def st_desc(g, m, b2):
            row = ((pair * PB + b2) * S2 + (s0 + g * C) * 2)
            return pltpu.make_async_copy(
                toks[m].at[pl.ds(b2 * HR, HR)], out_hbm.at[pl.ds(row, HR)],
                ssem[m],
            )

        def issue(g, m):
            tok_desc(g, m).start()
            pidx[m][...] = lax.iota(jnp.int32, LANES) + (
                2 * (s0 + POS_OFFSET + g * C)
            )
            pos_desc(g, m).start()

        issue(0, 0)
        issue(1, 1)

        def body(g, _):
            for m in range(NBUF):
                @pl.when(g % NBUF == m)
                def _slot(m=m):
                    tok_b, pos_b = toks[m], poss[m]
                    tok_desc(g, m).wait()
                    pos_desc(g, m).wait()

                    def add_row(hr, _):
                        def add_vec(j, _):
                            col = j * LANES
                            pvec = pos_b[hr, pl.ds(col, LANES)]
                            for b2 in range(PB):
                                plsc.addupdate(
                                    tok_b.at[b2 * HR + hr, pl.ds(col, LANES)],
                                    pvec,
                                )
                            return _
                        return lax.fori_loop(0, vecs, add_vec, _)

                    lax.fori_loop(0, HR, add_row, None)

                    for b2 in range(PB):
                        st_desc(g, m, b2).start()

                    @pl.when(g + 2 < NCH)
                    def _next(m=m):
                        m2 = (m + 2) % NBUF

                        @pl.when(g >= 1)
                        def _drain(m2=m2):
                            for b2 in range(PB):
                                st_desc(g - 1, m2, b2).wait()

                        issue(g + 2, m2)

            return _

        lax.fori_loop(0, NCH, body, None)

        for gl in range(NCH - 3, NCH):
            for b2 in range(PB):
                st_desc(gl, gl % NBUF, b2).wait()

    return emb


def kernel(token_ids, token_table, pos_table):
    B, S = token_ids.shape
    V, H = token_table.shape
    info = plsc.get_sparse_core_info()
    NC, NS = info.num_cores, info.num_subcores
    NW = NC * NS
    PB = 2
    PAIRS = B // PB
    WPP = NW // PAIRS
    SW = S // WPP
    C = 4
    t2 = token_ids * 2
    ids2 = jnp.stack([t2, t2 + 1], axis=-1)
    ids_perm = (
        ids2.reshape(PAIRS, PB, WPP, SW // C, C, 2)
        .transpose(0, 2, 3, 1, 4, 5)
        .reshape(-1)
    )
    emb = _make_kernel(B, S, V, H, NC, NS)
    out = emb(
        ids_perm,
        token_table.reshape(2 * V, H // 2),
        pos_table.reshape(-1, H // 2),
    )
    return out.reshape(B, S, H)
